# baseline (speedup 1.0000x reference)
import jax
import jax.numpy as jnp
from jax.experimental import pallas as pl
from jax.experimental.pallas import tpu as pltpu

B, T, C = 64, 2048, 64
MAXR, MAXC = 11, 11
NCOL = MAXR * MAXC
NPAD = 128
N = T * C


def _tc_body(src_ref, x_ref, o_ref, w_ref):
    b = pl.program_id(0)

    @pl.when(b == 0)
    def _build_w():
        iota_c = jax.lax.broadcasted_iota(jnp.int32, (NPAD, NPAD), 0)
        w_ref[...] = (iota_c == src_ref[0][None, :]).astype(jnp.float32)

    xb = x_ref[0]
    s1 = jnp.sum(xb)
    s2 = jnp.sum(xb * xb)
    mean = s1 / N
    var = (s2 - s1 * mean) / (N - 1)
    rstd = 1.0 / (jnp.sqrt(var) + 1e-6)
    w = w_ref[...]
    hi = xb.astype(jnp.bfloat16)
    lo = (xb - hi.astype(jnp.float32)).astype(jnp.bfloat16)
    dn = (((1,), (0,)), ((), ()))
    y = jax.lax.dot_general(hi, w.astype(jnp.bfloat16), dn,
                            preferred_element_type=jnp.float32)
    y = y + jax.lax.dot_general(lo, w.astype(jnp.bfloat16), dn,
                                preferred_element_type=jnp.float32)
    o_ref[0] = (y - mean) * rstd


def kernel(x, rearrange_idx, original_idx):
    src = jnp.zeros((MAXC,), jnp.int32).at[rearrange_idx].set(original_idx)
    full_src = jnp.tile(src, (NPAD // MAXC) + 1)[:NPAD].reshape(1, NPAD)
    xp = jnp.pad(x, ((0, 0), (0, 0), (0, NPAD - C)))

    y = pl.pallas_call(
        _tc_body,
        grid=(B,),
        in_specs=[
            pl.BlockSpec((1, NPAD), lambda b: (0, 0)),
            pl.BlockSpec((1, T, NPAD), lambda b: (b, 0, 0)),
        ],
        out_specs=pl.BlockSpec((1, T, NPAD), lambda b: (b, 0, 0)),
        out_shape=jax.ShapeDtypeStruct((B, T, NPAD), jnp.float32),
        scratch_shapes=[pltpu.VMEM((NPAD, NPAD), jnp.float32)],
    )(full_src, xp)
    return y[:, :, :NCOL].reshape(B, T, MAXR, MAXC)


# speedup vs baseline: 1.1614x; 1.1614x over previous
import jax
import jax.numpy as jnp
from jax.experimental import pallas as pl
from jax.experimental.pallas import tpu as pltpu

B, T, C = 64, 2048, 64
MAXR, MAXC = 11, 11
NCOL = MAXR * MAXC
NPAD = 128
N = T * C
SAMP = 4


def _tc_body(src_ref, x_ref, o_ref, w_ref):
    b = pl.program_id(0)

    @pl.when(b == 0)
    def _build_w():
        iota_c = jax.lax.broadcasted_iota(jnp.int32, (NPAD, NPAD), 0)
        w_ref[...] = (iota_c == src_ref[0][None, :]).astype(jnp.bfloat16)

    xb = x_ref[...]
    s1 = jnp.sum(xb, axis=(1, 2))
    s2 = jnp.sum(xb * xb, axis=(1, 2))
    mean = s1 / N
    var = (s2 - s1 * mean) / (N - 1)
    rstd = 1.0 / (jnp.sqrt(var) + 1e-6)
    xf = xb.reshape(SAMP * T, NPAD)
    w = w_ref[...]
    hi = xf.astype(jnp.bfloat16)
    lo = (xf - hi.astype(jnp.float32)).astype(jnp.bfloat16)
    dn = (((1,), (0,)), ((), ()))
    y = jax.lax.dot_general(hi, w, dn, preferred_element_type=jnp.float32)
    y = y + jax.lax.dot_general(lo, w, dn, preferred_element_type=jnp.float32)
    yb = y.reshape(SAMP, T, NPAD)
    o_ref[...] = (yb - mean[:, None, None]) * rstd[:, None, None]


def kernel(x, rearrange_idx, original_idx):
    src = jnp.zeros((MAXC,), jnp.int32).at[rearrange_idx].set(original_idx)
    full_src = jnp.tile(src, (NPAD // MAXC) + 1)[:NPAD].reshape(1, NPAD)
    xp = jnp.pad(x, ((0, 0), (0, 0), (0, NPAD - C)))

    y = pl.pallas_call(
        _tc_body,
        grid=(B // SAMP,),
        in_specs=[
            pl.BlockSpec((1, NPAD), lambda b: (0, 0)),
            pl.BlockSpec((SAMP, T, NPAD), lambda b: (b, 0, 0)),
        ],
        out_specs=pl.BlockSpec((SAMP, T, NPAD), lambda b: (b, 0, 0)),
        out_shape=jax.ShapeDtypeStruct((B, T, NPAD), jnp.float32),
        scratch_shapes=[pltpu.VMEM((NPAD, NPAD), jnp.bfloat16)],
    )(full_src, xp)
    return y[:, :, :NCOL].reshape(B, T, MAXR, MAXC)


# speedup vs baseline: 1.3051x; 1.1238x over previous
import jax
import jax.numpy as jnp
from jax.experimental import pallas as pl
from jax.experimental.pallas import tpu as pltpu

B, T, C = 64, 2048, 64
MAXR, MAXC = 11, 11
NCOL = MAXR * MAXC
NPAD = 128
N = T * C
SAMP = 4


def _tc_body(src_ref, x_ref, o_ref, w_ref):
    b = pl.program_id(0)

    @pl.when(b == 0)
    def _build_w():
        iota_c = jax.lax.broadcasted_iota(jnp.int32, (C, NPAD), 0)
        w_ref[...] = (iota_c == src_ref[0][None, :]).astype(jnp.bfloat16)

    xb = x_ref[...]
    s1 = jnp.sum(xb, axis=(1, 2))
    s2 = jnp.sum(xb * xb, axis=(1, 2))
    mean = s1 / N
    var = (s2 - s1 * mean) / (N - 1)
    rstd = 1.0 / (jnp.sqrt(var) + 1e-6)
    xf = xb.reshape(SAMP * T, C)
    w = w_ref[...]
    hi = xf.astype(jnp.bfloat16)
    lo = (xf - hi.astype(jnp.float32)).astype(jnp.bfloat16)
    dn = (((1,), (0,)), ((), ()))
    y = jax.lax.dot_general(hi, w, dn, preferred_element_type=jnp.float32)
    y = y + jax.lax.dot_general(lo, w, dn, preferred_element_type=jnp.float32)
    yb = y.reshape(SAMP, T, NPAD)
    o_ref[...] = (yb - mean[:, None, None]) * rstd[:, None, None]


def kernel(x, rearrange_idx, original_idx):
    src = jnp.zeros((MAXC,), jnp.int32).at[rearrange_idx].set(original_idx)
    full_src = jnp.tile(src, (NPAD // MAXC) + 1)[:NPAD].reshape(1, NPAD)
    xp = x

    y = pl.pallas_call(
        _tc_body,
        grid=(B // SAMP,),
        in_specs=[
            pl.BlockSpec((1, NPAD), lambda b: (0, 0)),
            pl.BlockSpec((SAMP, T, C), lambda b: (b, 0, 0)),
        ],
        out_specs=pl.BlockSpec((SAMP, T, NPAD), lambda b: (b, 0, 0)),
        out_shape=jax.ShapeDtypeStruct((B, T, NPAD), jnp.float32),
        scratch_shapes=[pltpu.VMEM((C, NPAD), jnp.bfloat16)],
    )(full_src, xp)
    return y[:, :, :NCOL].reshape(B, T, MAXR, MAXC)


# speedup vs baseline: 3.6137x; 2.7689x over previous
import jax
import jax.numpy as jnp
from jax.experimental import pallas as pl
from jax.experimental.pallas import tpu as pltpu

B, T, C = 64, 2048, 64
MAXR, MAXC = 11, 11
NCOL = MAXR * MAXC
CPAD = 16
N = T * C
GS = 8


def _gather_body(src_ref, x_ref, comp_ref, st_ref):
    xb = x_ref[...]
    s1 = jnp.sum(xb, axis=(1, 2))
    s2 = jnp.sum(xb * xb, axis=(1, 2))
    st_ref[0] = jnp.broadcast_to(s1[:, None], (GS, 128))
    st_ref[1] = jnp.broadcast_to(s2[:, None], (GS, 128))
    iota_c = jax.lax.broadcasted_iota(jnp.int32, (CPAD, C), 1)
    w = (iota_c == src_ref[0][:, None]).astype(jnp.bfloat16)
    hi = xb.astype(jnp.bfloat16)
    lo = (xb - hi.astype(jnp.float32)).astype(jnp.bfloat16)
    dn = (((1,), (1,)), ((), ()))
    y = jax.lax.dot_general(w, hi, dn, preferred_element_type=jnp.float32)
    y = y + jax.lax.dot_general(w, lo, dn, preferred_element_type=jnp.float32)
    comp_ref[...] = y


def _apply_body(st_ref, comp_ref, o_ref):
    s1 = st_ref[0, :, :1]
    s2 = st_ref[1, :, :1]
    mean = s1 * (1.0 / N)
    var = (s2 - s1 * mean) * (1.0 / (N - 1))
    rstd = 1.0 / (jnp.sqrt(var) + 1e-6)
    y = (comp_ref[0] - mean) * rstd
    o_ref[...] = jnp.broadcast_to(y[None, None], (MAXR, 1, B, T))


def kernel(x, rearrange_idx, original_idx):
    src = jnp.zeros((MAXC,), jnp.int32).at[rearrange_idx].set(original_idx)
    src16 = jnp.pad(src, (0, CPAD - MAXC)).reshape(1, CPAD)
    xt = jnp.transpose(x, (0, 2, 1))

    comp, st = pl.pallas_call(
        _gather_body,
        grid=(B // GS,),
        in_specs=[
            pl.BlockSpec((1, CPAD), lambda g: (0, 0)),
            pl.BlockSpec((GS, C, T), lambda g: (g, 0, 0)),
        ],
        out_specs=[
            pl.BlockSpec((CPAD, GS, T), lambda g: (0, g, 0)),
            pl.BlockSpec((2, GS, 128), lambda g: (0, g, 0)),
        ],
        out_shape=[
            jax.ShapeDtypeStruct((CPAD, B, T), jnp.float32),
            jax.ShapeDtypeStruct((2, B, 128), jnp.float32),
        ],
    )(src16, xt)

    yo = pl.pallas_call(
        _apply_body,
        grid=(MAXC,),
        in_specs=[
            pl.BlockSpec((2, B, 128), lambda c: (0, 0, 0)),
            pl.BlockSpec((1, B, T), lambda c: (c, 0, 0)),
        ],
        out_specs=pl.BlockSpec((MAXR, 1, B, T), lambda c: (0, c, 0, 0)),
        out_shape=jax.ShapeDtypeStruct((MAXR, MAXC, B, T), jnp.float32),
    )(st, comp)

    return jnp.transpose(yo, (2, 3, 0, 1))
